# baseline (speedup 1.0000x reference)
import functools

import jax
import jax.numpy as jnp
from jax import lax
from jax.experimental import pallas as pl
from jax.experimental.pallas import tpu as pltpu
from jax.experimental.pallas import tpu_sc as plsc

NC = 2
NS = 16
NW = NC * NS
L = 16
F = 16
CHUNK = 128


def _build(batch, n_rows):
    n_per = batch // NW
    n_chunks = n_per // CHUNK
    n_groups = n_per // L
    mesh = plsc.VectorSubcoreMesh(core_axis_name="c", subcore_axis_name="s")

    @functools.partial(
        pl.kernel,
        out_type=jax.ShapeDtypeStruct((batch,), jnp.float32),
        mesh=mesh,
        compiler_params=pltpu.CompilerParams(
            needs_layout_passes=False, use_tc_tiling_on_sc=False
        ),
        scratch_types=[
            pltpu.VMEM((F, n_chunks, CHUNK), jnp.int32),
            pltpu.VMEM((F, n_chunks, CHUNK), jnp.int32),
            pltpu.VMEM((F, n_chunks, CHUNK), jnp.float32),
            pltpu.VMEM((F, n_chunks, CHUNK), jnp.float32),
            pltpu.VMEM((n_per,), jnp.float32),
            pltpu.SemaphoreType.DMA,
            pltpu.SemaphoreType.DMA,
        ],
    )
    def mf(bi_hbm, si_hbm, uef_hbm, ief_hbm, out_hbm,
           bi_v, si_v, gu_v, gi_v, o_v, sem_u, sem_i):
        wid = lax.axis_index("s") * NC + lax.axis_index("c")
        pltpu.sync_copy(bi_hbm.at[wid], bi_v)
        pltpu.sync_copy(si_hbm.at[wid], si_v)

        copies = []
        for f in range(F):
            for c in range(n_chunks):
                copies.append(
                    pltpu.async_copy(uef_hbm.at[bi_v.at[f, c]], gu_v.at[f, c], sem_u)
                )
                copies.append(
                    pltpu.async_copy(ief_hbm.at[si_v.at[f, c]], gi_v.at[f, c], sem_i)
                )
        for cp in copies:
            cp.wait()

        for g in range(n_groups):
            c, l = g // 8, (g % 8) * L
            acc = gu_v[0, c, pl.ds(l, L)] * gi_v[0, c, pl.ds(l, L)]
            for f in range(1, F):
                acc = acc + gu_v[f, c, pl.ds(l, L)] * gi_v[f, c, pl.ds(l, L)]
            o_v[pl.ds(g * L, L)] = acc

        base = pl.multiple_of(wid * n_per, n_per)
        pltpu.sync_copy(o_v, out_hbm.at[pl.ds(base, n_per)])

    return mf


_mf = _build(16384, 1000000)


def kernel(b, s, user_embedding, item_embedding):
    batch = b.shape[0]
    n_rows = user_embedding.shape[0]
    uef = user_embedding.T.reshape(-1)
    ief = item_embedding.T.reshape(-1)
    feat = (jnp.arange(F, dtype=jnp.int32) * n_rows).reshape(1, F, 1, 1)
    bidx = b.reshape(NW, 1, batch // NW // CHUNK, CHUNK) + feat
    sidx = s.reshape(NW, 1, batch // NW // CHUNK, CHUNK) + feat
    return _mf(bidx, sidx, uef, ief)


# speedup vs baseline: 1.3951x; 1.3951x over previous
import functools

import jax
import jax.numpy as jnp
from jax import lax
from jax.experimental import pallas as pl
from jax.experimental.pallas import tpu as pltpu
from jax.experimental.pallas import tpu_sc as plsc

NC = 2
NS = 16
NW = NC * NS
L = 16
F = 16
FP = F // 2
CHUNK = 128


def _build(batch):
    n_per = batch // NW
    n_chunks = n_per // CHUNK
    n_groups = n_per // L
    mesh = plsc.VectorSubcoreMesh(core_axis_name="c", subcore_axis_name="s")

    @functools.partial(
        pl.kernel,
        out_type=jax.ShapeDtypeStruct((batch,), jnp.float32),
        mesh=mesh,
        compiler_params=pltpu.CompilerParams(
            needs_layout_passes=False, use_tc_tiling_on_sc=False
        ),
        scratch_types=[
            pltpu.VMEM((n_chunks, CHUNK), jnp.int32),
            pltpu.VMEM((n_chunks, CHUNK), jnp.int32),
            pltpu.VMEM((n_per, FP), jnp.int32),
            pltpu.VMEM((n_per, FP), jnp.int32),
            pltpu.VMEM((n_per,), jnp.float32),
            pltpu.SemaphoreType.DMA,
            pltpu.SemaphoreType.DMA,
        ],
    )
    def mf(bi_hbm, si_hbm, ue_hbm, ie_hbm, out_hbm,
           bi_v, si_v, gu_v, gi_v, o_v, sem_u, sem_i):
        wid = lax.axis_index("s") * NC + lax.axis_index("c")
        pltpu.sync_copy(bi_hbm.at[wid], bi_v)
        pltpu.sync_copy(si_hbm.at[wid], si_v)

        copies = []
        for c in range(n_chunks):
            dst = pl.ds(c * CHUNK, CHUNK)
            copies.append(pltpu.async_copy(ue_hbm.at[bi_v.at[c]], gu_v.at[dst], sem_u))
            copies.append(pltpu.async_copy(ie_hbm.at[si_v.at[c]], gi_v.at[dst], sem_i))
        for cp in copies:
            cp.wait()

        lanes = lax.iota(jnp.int32, L)
        for g in range(n_groups):
            rows = lanes + g * L
            acc = jnp.zeros((L,), jnp.float32)
            for p in range(FP):
                col = jnp.full((L,), p, jnp.int32)
                uw = plsc.load_gather(gu_v, [rows, col])
                iw = plsc.load_gather(gi_v, [rows, col])
                ua, ub = plsc.unpack(
                    plsc.bitcast(uw, jnp.bfloat16),
                    format=plsc.PackFormat.INTERLEAVED,
                )
                ia, ib = plsc.unpack(
                    plsc.bitcast(iw, jnp.bfloat16),
                    format=plsc.PackFormat.INTERLEAVED,
                )
                acc = acc + ua * ia + ub * ib
            o_v[pl.ds(g * L, L)] = acc

        base = pl.multiple_of(wid * n_per, n_per)
        pltpu.sync_copy(o_v, out_hbm.at[pl.ds(base, n_per)])

    return mf


_mf = _build(16384)


def kernel(b, s, user_embedding, item_embedding):
    batch = b.shape[0]
    n_rows = user_embedding.shape[0]
    ue_bf = user_embedding.astype(jnp.bfloat16)
    ie_bf = item_embedding.astype(jnp.bfloat16)
    ue_i32 = lax.bitcast_convert_type(ue_bf.reshape(n_rows, FP, 2), jnp.int32)
    ie_i32 = lax.bitcast_convert_type(ie_bf.reshape(n_rows, FP, 2), jnp.int32)
    n_chunks = batch // NW // CHUNK
    b3 = b.reshape(NW, n_chunks, CHUNK)
    s3 = s.reshape(NW, n_chunks, CHUNK)
    return _mf(b3, s3, ue_i32, ie_i32)


# speedup vs baseline: 3.1769x; 2.2772x over previous
import functools

import jax
import jax.numpy as jnp
from jax import lax
from jax.experimental import pallas as pl
from jax.experimental.pallas import tpu as pltpu
from jax.experimental.pallas import tpu_sc as plsc

NC = 2
NS = 16
NW = NC * NS
L = 16
F = 16
CHUNK = 128


def _build(batch):
    n_per = batch // NW
    n_chunks = n_per // CHUNK
    n_groups = n_per // L
    mesh = plsc.VectorSubcoreMesh(core_axis_name="c", subcore_axis_name="s")

    @functools.partial(
        pl.kernel,
        out_type=jax.ShapeDtypeStruct((batch,), jnp.float32),
        mesh=mesh,
        compiler_params=pltpu.CompilerParams(
            needs_layout_passes=False, use_tc_tiling_on_sc=False
        ),
        scratch_types=[
            pltpu.VMEM((n_chunks, CHUNK), jnp.int32),
            pltpu.VMEM((n_chunks, CHUNK), jnp.int32),
            pltpu.VMEM((n_per, F), jnp.float32),
            pltpu.VMEM((n_per, F), jnp.float32),
            pltpu.VMEM((n_per,), jnp.float32),
            pltpu.SemaphoreType.DMA,
            pltpu.SemaphoreType.DMA,
        ],
    )
    def mf(b_hbm, s_hbm, ue_hbm, ie_hbm, out_hbm,
           bi_v, si_v, u_v, i_v, o_v, sem_u, sem_i):
        wid = lax.axis_index("s") * NC + lax.axis_index("c")
        pltpu.sync_copy(b_hbm.at[wid], bi_v)
        pltpu.sync_copy(s_hbm.at[wid], si_v)

        copies = []
        for j in range(n_chunks):
            dst = pl.ds(j * CHUNK, CHUNK)
            copies.append(pltpu.async_copy(ue_hbm.at[bi_v.at[j]], u_v.at[dst], sem_u))
            copies.append(pltpu.async_copy(ie_hbm.at[si_v.at[j]], i_v.at[dst], sem_i))
        for c in copies:
            c.wait()

        lanes = lax.iota(jnp.int32, L)
        for g in range(n_groups):
            rows = lanes + g * L
            acc = jnp.zeros((L,), jnp.float32)
            for f in range(F):
                col = jnp.full((L,), f, jnp.int32)
                acc = acc + (plsc.load_gather(u_v, [rows, col])
                             * plsc.load_gather(i_v, [rows, col]))
            o_v[pl.ds(g * L, L)] = acc

        base = pl.multiple_of(wid * n_per, n_per)
        pltpu.sync_copy(o_v, out_hbm.at[pl.ds(base, n_per)])

    return mf


_mf = _build(16384)


def kernel(b, s, user_embedding, item_embedding):
    batch = b.shape[0]
    b3 = b.reshape(NW, batch // NW // CHUNK, CHUNK)
    s3 = s.reshape(NW, batch // NW // CHUNK, CHUNK)
    return _mf(b3, s3, user_embedding, item_embedding)
